# baseline (speedup 1.0000x reference)
import functools

import jax
import jax.numpy as jnp
from jax import lax
from jax.experimental import pallas as pl
from jax.experimental.pallas import tpu as pltpu
from jax.experimental.pallas import tpu_sc as plsc

NUM_CORES = 2
NUM_SUBCORES = 16
LANES = 16
NW = NUM_CORES * NUM_SUBCORES

B = 16384
D = 32
BPW = B // NW
CHUNK = 128
NCHUNK = BPW // CHUNK


def _body(uidx_hbm, iidx_hbm, uemb_hbm, iemb_hbm, ubias_hbm, ibias_hbm,
          gbias_hbm, out_hbm,
          idx_u, idx_i, urows, irows, ubv, ibv, gbv, outv, sem):
    wid = lax.axis_index("s") * NUM_CORES + lax.axis_index("c")
    base = wid * BPW

    for j in range(NCHUNK):
        pltpu.sync_copy(uidx_hbm.at[pl.ds(base + j * CHUNK, CHUNK)], idx_u.at[j])
        pltpu.sync_copy(iidx_hbm.at[pl.ds(base + j * CHUNK, CHUNK)], idx_i.at[j])
    pltpu.sync_copy(gbias_hbm, gbv.at[pl.ds(0, 1)])

    handles = []
    for j in range(NCHUNK):
        sl = pl.ds(j * CHUNK, CHUNK)
        handles.append(pltpu.async_copy(uemb_hbm.at[idx_u.at[j]], urows.at[sl], sem))
        handles.append(pltpu.async_copy(iemb_hbm.at[idx_i.at[j]], irows.at[sl], sem))
        handles.append(pltpu.async_copy(ubias_hbm.at[idx_u.at[j]], ubv.at[sl], sem))
        handles.append(pltpu.async_copy(ibias_hbm.at[idx_i.at[j]], ibv.at[sl], sem))
    for h in handles:
        h.wait()

    gb = gbv[...][0]
    iota = lax.broadcasted_iota(jnp.int32, (LANES,), 0)

    for g in range(BPW // LANES):
        rvec = iota + g * LANES
        acc = ubv[pl.ds(g * LANES, LANES)] + ibv[pl.ds(g * LANES, LANES)] + gb

        def dbody(dd, a):
            dsplat = jnp.full((LANES,), dd, jnp.int32)
            return a + (plsc.load_gather(urows, [rvec, dsplat])
                        * plsc.load_gather(irows, [rvec, dsplat]))

        acc = lax.fori_loop(0, D, dbody, acc)
        outv[pl.ds(g * LANES, LANES)] = acc

    pltpu.sync_copy(outv, out_hbm.at[pl.ds(base, BPW)])


@jax.jit
def _run(user_idx, item_idx, user_embeddings, item_embeddings,
         user_bias, item_bias, global_bias):
    mesh = plsc.VectorSubcoreMesh(
        core_axis_name="c", subcore_axis_name="s",
        num_cores=NUM_CORES, num_subcores=NUM_SUBCORES)
    f = functools.partial(
        pl.kernel,
        out_type=jax.ShapeDtypeStruct((B,), jnp.float32),
        mesh=mesh,
        compiler_params=pltpu.CompilerParams(
            needs_layout_passes=False, use_tc_tiling_on_sc=False),
        scratch_types=[
            pltpu.VMEM((NCHUNK, CHUNK), jnp.int32),
            pltpu.VMEM((NCHUNK, CHUNK), jnp.int32),
            pltpu.VMEM((BPW, D), jnp.float32),
            pltpu.VMEM((BPW, D), jnp.float32),
            pltpu.VMEM((BPW,), jnp.float32),
            pltpu.VMEM((BPW,), jnp.float32),
            pltpu.VMEM((LANES,), jnp.float32),
            pltpu.VMEM((BPW,), jnp.float32),
            pltpu.SemaphoreType.DMA,
        ],
    )(_body)
    return f(user_idx, item_idx, user_embeddings, item_embeddings,
             user_bias, item_bias, global_bias)


def kernel(user_idx, item_idx, user_embeddings, item_embeddings,
           user_bias, item_bias, global_bias):
    return _run(user_idx.astype(jnp.int32), item_idx.astype(jnp.int32),
                user_embeddings, item_embeddings,
                user_bias.reshape(-1), item_bias.reshape(-1), global_bias)
